# baseline (speedup 1.0000x reference)
import functools

import jax
import jax.numpy as jnp
from jax import lax
from jax.experimental import pallas as pl
from jax.experimental.pallas import tpu as pltpu
from jax.experimental.pallas import tpu_sc as plsc

N_NODES = 10000
N_HYEDGES = 10000
E = 320000
D = 128
N_TARGET = 16

NC = 2
NS = 16
NW = NC * NS

CH = 80
NCHP = 128
PADN = NW * NCHP * CH - E
CH0 = 170
CH1 = 2 * NCHP - CH0

NPAD = 10240
ROWS_PER_TILE = NPAD // NS

CW = 128
NCHC = 256

_mesh = plsc.VectorSubcoreMesh(core_axis_name="c", subcore_axis_name="s")


@functools.partial(
    pl.kernel,
    out_type=jax.ShapeDtypeStruct((NC * NPAD, D), jnp.float32),
    mesh=_mesh,
    scratch_types=[
        pltpu.VMEM((CH0 * CH,), jnp.int32),
        pltpu.VMEM((CH,), jnp.int32),
        pltpu.VMEM((CH,), jnp.int32),
        pltpu.VMEM((CH,), jnp.int32),
        pltpu.VMEM((CH,), jnp.int32),
        pltpu.VMEM((CH, D), jnp.float32),
        pltpu.VMEM((CH, D), jnp.float32),
        pltpu.VMEM_SHARED((NPAD, D), jnp.float32),
        pltpu.SemaphoreType.DMA,
        pltpu.SemaphoreType.DMA,
        pltpu.SemaphoreType.DMA,
        pltpu.SemaphoreType.DMA,
        pltpu.SemaphoreType.DMA,
        pltpu.SemaphoreType.DMA,
    ],
)
def _sc_seg_sum(table, src, dst, zeros, out,
                dall, idx_a, idx_b, db_a, db_b, rows_a, rows_b, acc,
                gs_a, gs_b, ss_a, ss_b, is_a, is_b):
    c = lax.axis_index("c")
    s = lax.axis_index("s")
    row0 = s * ROWS_PER_TILE

    pltpu.sync_copy(zeros, acc.at[pl.ds(row0, ROWS_PER_TILE)])

    def dwait(sem, buf):
        pltpu.make_async_copy(zeros.at[pl.ds(0, CH)], buf, sem).wait()

    def dwait_i(sem, buf):
        pltpu.make_async_copy(src.at[pl.ds(0, CH)], buf, sem).wait()

    @pl.when(c == 0)
    def _():
        pltpu.sync_copy(dst.at[pl.ds(s * CH0 * CH, CH0 * CH)], dall)
    plsc.subcore_barrier()

    @pl.when(c == 0)
    def _():
        base = s * CH0 * CH

        def regcopy(g, db):
            for k in range(CH // 16):
                db[pl.ds(k * 16, 16)] = dall[pl.ds(g * CH + k * 16, 16)]

        pltpu.sync_copy(src.at[pl.ds(base, CH)], idx_a)
        pltpu.async_copy(src.at[pl.ds(base + CH, CH)], idx_b, is_b)
        pltpu.async_copy(table.at[idx_a], rows_a, gs_a)

        def body(i, carry):
            g = 2 * i
            regcopy(g, db_a)
            dwait(gs_a, rows_a)

            @pl.when(i > 0)
            def _():
                dwait(ss_b, rows_b)

            dwait_i(is_b, idx_b)
            pltpu.async_copy(table.at[idx_b], rows_b, gs_b)
            pltpu.async_copy(rows_a, acc.at[db_a], ss_a, add=True)

            @pl.when(i < CH0 // 2 - 1)
            def _():
                pltpu.async_copy(src.at[pl.ds(base + (g + 2) * CH, CH)],
                                 idx_a, is_a)

            regcopy(g + 1, db_b)
            dwait(gs_b, rows_b)
            dwait(ss_a, rows_a)

            @pl.when(i < CH0 // 2 - 1)
            def _():
                dwait_i(is_a, idx_a)
                pltpu.async_copy(table.at[idx_a], rows_a, gs_a)

            @pl.when(i < CH0 // 2 - 1)
            def _():
                pltpu.async_copy(src.at[pl.ds(base + (g + 3) * CH, CH)],
                                 idx_b, is_b)

            pltpu.async_copy(rows_b, acc.at[db_b], ss_b, add=True)
            return carry

        lax.fori_loop(0, CH0 // 2, body, 0)
        dwait(ss_b, rows_b)

    @pl.when(c == 1)
    def _():
        base = (NS * CH0 + s * CH1) * CH

        def body(g, carry):
            off = base + g * CH
            pltpu.sync_copy(src.at[pl.ds(off, CH)], idx_a)
            pltpu.sync_copy(dst.at[pl.ds(off, CH)], db_a)
            pltpu.async_copy(table.at[idx_a], rows_a, gs_a).wait()
            pltpu.sync_copy(rows_a, acc.at[db_a], add=True)
            return carry

        lax.fori_loop(0, CH1, body, 0)

    plsc.subcore_barrier()
    pltpu.sync_copy(acc.at[pl.ds(row0, ROWS_PER_TILE)],
                    out.at[pl.ds(c * NPAD + row0, ROWS_PER_TILE)])


@functools.partial(
    pl.kernel,
    out_type=jax.ShapeDtypeStruct((NC * NPAD, CW), jnp.float32),
    mesh=_mesh,
    scratch_types=[
        pltpu.VMEM((NCHC, CH), jnp.int32),
        pltpu.VMEM((CH, CW), jnp.float32),
        pltpu.VMEM_SHARED((NPAD, CW), jnp.float32),
        pltpu.SemaphoreType.DMA,
        pltpu.SemaphoreType.DMA,
        pltpu.SemaphoreType.DMA,
        pltpu.SemaphoreType.DMA,
    ],
)
def _sc_counts(cidx4, ones, zeros, out, cidxv, ones_v, cnt, s0, s1, s2, s3):
    c = lax.axis_index("c")
    s = lax.axis_index("s")
    row0 = s * ROWS_PER_TILE

    pltpu.sync_copy(cidx4.at[c, s], cidxv)
    pltpu.sync_copy(ones, ones_v)
    pltpu.sync_copy(zeros, cnt.at[pl.ds(row0, ROWS_PER_TILE)])
    plsc.subcore_barrier()

    sems = (s0, s1, s2, s3)

    def dwait(sem):
        pltpu.make_async_copy(zeros.at[pl.ds(0, CH)], ones_v, sem).wait()

    def body(i, carry):
        for b in range(4):
            g = 4 * i + b
            sem = sems[b]

            @pl.when(i > 0)
            def _(sem=sem):
                dwait(sem)

            pltpu.async_copy(ones_v, cnt.at[cidxv.at[g]], sem, add=True)
        return carry

    lax.fori_loop(0, NCHC // 4, body, 0)
    for b in range(4):
        dwait(sems[b])

    plsc.subcore_barrier()
    pltpu.sync_copy(cnt.at[pl.ds(row0, ROWS_PER_TILE)],
                    out.at[pl.ds(c * NPAD + row0, ROWS_PER_TILE)])


BR = 1024
GRID = 10


def _leaky(x):
    return jnp.where(x >= 0, x, 0.01 * x)


def _spec(jmap):
    return pl.BlockSpec((BR, D), jmap)


def _mm_body(x_ref, w_ref, o_ref):
    o_ref[...] = jnp.dot(x_ref[...], w_ref[...],
                         preferred_element_type=jnp.float32)


def _tc_matmul(x, w):
    return pl.pallas_call(
        _mm_body,
        grid=(GRID,),
        in_specs=[_spec(lambda i: (i, 0)),
                  pl.BlockSpec((D, D), lambda i: (0, 0))],
        out_specs=_spec(lambda i: (i, 0)),
        out_shape=jax.ShapeDtypeStruct((N_NODES, D), jnp.float32),
    )(x, w)


def _comb_body(p0_ref, p1_ref, cnt_ref, o_ref):
    den = jnp.maximum(cnt_ref[...][:, :1], 1.0)
    o_ref[...] = (p0_ref[...] + p1_ref[...]) / den


def _tc_combine(p, counts):
    return pl.pallas_call(
        _comb_body,
        grid=(GRID,),
        in_specs=[_spec(lambda i: (i, 0)),
                  _spec(lambda i: (i + 10, 0)),
                  pl.BlockSpec((BR, CW), lambda i: (i + 10, 0))],
        out_specs=_spec(lambda i: (i, 0)),
        out_shape=jax.ShapeDtypeStruct((N_NODES, D), jnp.float32),
    )(p, p, counts)


def _comb_mm_body(p0_ref, p1_ref, cnt_ref, b_ref, w_ref, o_ref):
    den = jnp.maximum(cnt_ref[...][:, :1], 1.0)
    h = _leaky((p0_ref[...] + p1_ref[...]) / den + b_ref[...])
    o_ref[...] = jnp.dot(h, w_ref[...], preferred_element_type=jnp.float32)


def _tc_combine_matmul(p, counts, b, w):
    return pl.pallas_call(
        _comb_mm_body,
        grid=(GRID,),
        in_specs=[_spec(lambda i: (i, 0)),
                  _spec(lambda i: (i + 10, 0)),
                  pl.BlockSpec((BR, CW), lambda i: (i, 0)),
                  pl.BlockSpec((1, D), lambda i: (0, 0)),
                  pl.BlockSpec((D, D), lambda i: (0, 0))],
        out_specs=_spec(lambda i: (i, 0)),
        out_shape=jax.ShapeDtypeStruct((N_NODES, D), jnp.float32),
    )(p, p, counts, b, w)


def _final_body(p0_ref, p1_ref, cnt_ref, b_ref, wf_ref, bf_ref,
                feats_ref, fp_ref, out_ref):
    i = pl.program_id(0)
    den = jnp.maximum(cnt_ref[...][:, :1], 1.0)
    h = _leaky((p0_ref[...] + p1_ref[...]) / den + b_ref[...])
    feats_ref[...] = h
    rid = lax.broadcasted_iota(jnp.int32, (BR, 1), 0) + i * BR
    part = jnp.sum(jnp.where(rid < N_NODES, h, 0.0), axis=0, keepdims=True)

    @pl.when(i == 0)
    def _():
        fp_ref[...] = jnp.zeros_like(fp_ref)
        out_ref[...] = jnp.zeros_like(out_ref)

    fp_ref[...] += part

    @pl.when(i == GRID - 1)
    def _():
        fp = fp_ref[...] / float(N_NODES)
        fp_ref[...] = fp
        logits = jnp.dot(fp, wf_ref[...],
                         preferred_element_type=jnp.float32) + bf_ref[...]
        out_ref[...] = 1.0 / (1.0 + jnp.exp(-logits))


def _tc_final(p, counts, b, wf, bf):
    return pl.pallas_call(
        _final_body,
        grid=(GRID,),
        in_specs=[_spec(lambda i: (i, 0)),
                  _spec(lambda i: (i + 10, 0)),
                  pl.BlockSpec((BR, CW), lambda i: (i, 0)),
                  pl.BlockSpec((1, D), lambda i: (0, 0)),
                  pl.BlockSpec((D, N_TARGET), lambda i: (0, 0)),
                  pl.BlockSpec((1, N_TARGET), lambda i: (0, 0))],
        out_specs=[_spec(lambda i: (i, 0)),
                   pl.BlockSpec((1, D), lambda i: (0, 0)),
                   pl.BlockSpec((1, N_TARGET), lambda i: (0, 0))],
        out_shape=[jax.ShapeDtypeStruct((N_NODES, D), jnp.float32),
                   jax.ShapeDtypeStruct((1, D), jnp.float32),
                   jax.ShapeDtypeStruct((1, N_TARGET), jnp.float32)],
    )(p, p, counts, b, wf, bf)


def kernel(x, H, W1, b1, W2, b2, Wf, bf):
    node_idx = H[0]
    hyedge_idx = H[1]
    pad_src = jnp.zeros((PADN,), jnp.int32)
    pad_dst = jnp.full((PADN,), N_NODES, jnp.int32)
    node_src1 = jnp.concatenate([node_idx, pad_src])
    hy_src1 = jnp.concatenate([hyedge_idx, pad_src])
    node_dst1 = jnp.concatenate([node_idx, pad_dst])
    hy_dst1 = jnp.concatenate([hyedge_idx, pad_dst])
    cidx4 = jnp.pad(H.reshape(2, NS, NCHC - 6, CH),
                    ((0, 0), (0, 0), (0, 6), (0, 0)),
                    constant_values=N_NODES)

    zeros_d = jnp.zeros((ROWS_PER_TILE, D), jnp.float32)
    ones_c = jnp.ones((CH, CW), jnp.float32)

    counts = _sc_counts(cidx4, ones_c, zeros_d)

    y1 = _tc_matmul(x, W1)
    ep = _sc_seg_sum(y1, node_src1, hy_dst1, zeros_d)
    e1 = _tc_combine(ep, counts)
    np_ = _sc_seg_sum(e1, hy_src1, node_dst1, zeros_d)
    y2 = _tc_combine_matmul(np_, counts, b1.reshape(1, D), W2)
    ep2 = _sc_seg_sum(y2, node_src1, hy_dst1, zeros_d)
    e2 = _tc_combine(ep2, counts)
    np2 = _sc_seg_sum(e2, hy_src1, node_dst1, zeros_d)
    feats, fp, out = _tc_final(np2, counts, b2.reshape(1, D),
                               Wf, bf.reshape(1, N_TARGET))
    return (out.reshape(N_TARGET), feats, fp)


# speedup vs baseline: 1.5293x; 1.5293x over previous
import functools

import jax
import jax.numpy as jnp
from jax import lax
from jax.experimental import pallas as pl
from jax.experimental.pallas import tpu as pltpu
from jax.experimental.pallas import tpu_sc as plsc

N_NODES = 10000
N_HYEDGES = 10000
E = 320000
D = 128
N_TARGET = 16

NC = 2
NS = 16
NW = NC * NS
EPW = E // NW
CH = 80
NCHUNK = EPW // CH

NPAD = 10240
ROWS_PER_TILE = NPAD // NS

CW = 128

_mesh = plsc.VectorSubcoreMesh(core_axis_name="c", subcore_axis_name="s")


@functools.partial(
    pl.kernel,
    out_type=jax.ShapeDtypeStruct((NC * NPAD, D), jnp.float32),
    mesh=_mesh,
    scratch_types=[
        pltpu.VMEM((CH,), jnp.int32),
        pltpu.VMEM((CH,), jnp.int32),
        pltpu.VMEM((CH, D), jnp.float32),
        pltpu.VMEM_SHARED((NPAD, D), jnp.float32),
        pltpu.SemaphoreType.DMA,
    ],
)
def _sc_seg_sum(table, src, dst, zeros, out, idx_s, idx_d, rows, acc, sem):
    c = lax.axis_index("c")
    s = lax.axis_index("s")
    wid = c * NS + s
    base = wid * EPW

    row0 = s * ROWS_PER_TILE
    pltpu.sync_copy(zeros, acc.at[pl.ds(row0, ROWS_PER_TILE)])
    plsc.subcore_barrier()

    def body(g, carry):
        off = base + g * CH
        pltpu.sync_copy(src.at[pl.ds(off, CH)], idx_s)
        pltpu.sync_copy(dst.at[pl.ds(off, CH)], idx_d)
        pltpu.async_copy(table.at[idx_s], rows, sem).wait()
        pltpu.sync_copy(rows, acc.at[idx_d], add=True)
        return carry

    lax.fori_loop(0, NCHUNK, body, 0)

    plsc.subcore_barrier()
    pltpu.sync_copy(acc.at[pl.ds(row0, ROWS_PER_TILE)],
                    out.at[pl.ds(c * NPAD + row0, ROWS_PER_TILE)])


EPT_CNT = E // NS
NCHUNK_CNT = EPT_CNT // CH


@functools.partial(
    pl.kernel,
    out_type=jax.ShapeDtypeStruct((NC * NPAD, CW), jnp.float32),
    mesh=_mesh,
    scratch_types=[
        pltpu.VMEM((CH,), jnp.int32),
        pltpu.VMEM((CH, CW), jnp.float32),
        pltpu.VMEM_SHARED((NPAD, CW), jnp.float32),
    ],
)
def _sc_counts(hflat, ones, zeros, out, idx_d, ones_v, cnt):
    c = lax.axis_index("c")
    s = lax.axis_index("s")

    row0 = s * ROWS_PER_TILE
    pltpu.sync_copy(zeros, cnt.at[pl.ds(row0, ROWS_PER_TILE)])
    pltpu.sync_copy(ones, ones_v)
    plsc.subcore_barrier()

    base = c * E + s * EPT_CNT

    def body(g, carry):
        off = base + g * CH
        pltpu.sync_copy(hflat.at[pl.ds(off, CH)], idx_d)
        pltpu.sync_copy(ones_v, cnt.at[idx_d], add=True)
        return carry

    lax.fori_loop(0, NCHUNK_CNT, body, 0)

    plsc.subcore_barrier()
    pltpu.sync_copy(cnt.at[pl.ds(row0, ROWS_PER_TILE)],
                    out.at[pl.ds(c * NPAD + row0, ROWS_PER_TILE)])


BR = 1000
GRID = N_NODES // BR


def _mm_body(x_ref, w_ref, o_ref):
    o_ref[...] = jnp.dot(x_ref[...], w_ref[...],
                         preferred_element_type=jnp.float32)


def _tc_matmul(x, w):
    return pl.pallas_call(
        _mm_body,
        grid=(GRID,),
        in_specs=[pl.BlockSpec((BR, D), lambda i: (i, 0)),
                  pl.BlockSpec((D, D), lambda i: (0, 0))],
        out_specs=pl.BlockSpec((BR, D), lambda i: (i, 0)),
        out_shape=jax.ShapeDtypeStruct((N_NODES, D), jnp.float32),
    )(x, w)


def _comb_body(p0_ref, p1_ref, cnt_ref, o_ref):
    den = jnp.maximum(cnt_ref[...][:, :1], 1.0)
    o_ref[...] = (p0_ref[...] + p1_ref[...]) / den


def _tc_combine(p0, p1, cnt):
    return pl.pallas_call(
        _comb_body,
        grid=(GRID,),
        in_specs=[pl.BlockSpec((BR, D), lambda i: (i, 0)),
                  pl.BlockSpec((BR, D), lambda i: (i, 0)),
                  pl.BlockSpec((BR, CW), lambda i: (i, 0))],
        out_specs=pl.BlockSpec((BR, D), lambda i: (i, 0)),
        out_shape=jax.ShapeDtypeStruct((N_NODES, D), jnp.float32),
    )(p0, p1, cnt)


def _leaky(x):
    return jnp.where(x >= 0, x, 0.01 * x)


def _comb_mm_body(p0_ref, p1_ref, cnt_ref, b_ref, w_ref, o_ref):
    den = jnp.maximum(cnt_ref[...][:, :1], 1.0)
    h = _leaky((p0_ref[...] + p1_ref[...]) / den + b_ref[...])
    o_ref[...] = jnp.dot(h, w_ref[...], preferred_element_type=jnp.float32)


def _tc_combine_matmul(p0, p1, cnt, b, w):
    return pl.pallas_call(
        _comb_mm_body,
        grid=(GRID,),
        in_specs=[pl.BlockSpec((BR, D), lambda i: (i, 0)),
                  pl.BlockSpec((BR, D), lambda i: (i, 0)),
                  pl.BlockSpec((BR, CW), lambda i: (i, 0)),
                  pl.BlockSpec((1, D), lambda i: (0, 0)),
                  pl.BlockSpec((D, D), lambda i: (0, 0))],
        out_specs=pl.BlockSpec((BR, D), lambda i: (i, 0)),
        out_shape=jax.ShapeDtypeStruct((N_NODES, D), jnp.float32),
    )(p0, p1, cnt, b, w)


def _final_body(p0_ref, p1_ref, cnt_ref, b_ref, wf_ref, bf_ref,
                feats_ref, fp_ref, out_ref):
    i = pl.program_id(0)
    den = jnp.maximum(cnt_ref[...][:, :1], 1.0)
    h = _leaky((p0_ref[...] + p1_ref[...]) / den + b_ref[...])
    feats_ref[...] = h
    part = jnp.sum(h, axis=0, keepdims=True)

    @pl.when(i == 0)
    def _():
        fp_ref[...] = jnp.zeros_like(fp_ref)
        out_ref[...] = jnp.zeros_like(out_ref)

    fp_ref[...] += part

    @pl.when(i == GRID - 1)
    def _():
        fp = fp_ref[...] / float(N_NODES)
        fp_ref[...] = fp
        logits = jnp.dot(fp, wf_ref[...],
                         preferred_element_type=jnp.float32) + bf_ref[...]
        out_ref[...] = 1.0 / (1.0 + jnp.exp(-logits))


def _tc_final(p0, p1, cnt, b, wf, bf):
    return pl.pallas_call(
        _final_body,
        grid=(GRID,),
        in_specs=[pl.BlockSpec((BR, D), lambda i: (i, 0)),
                  pl.BlockSpec((BR, D), lambda i: (i, 0)),
                  pl.BlockSpec((BR, CW), lambda i: (i, 0)),
                  pl.BlockSpec((1, D), lambda i: (0, 0)),
                  pl.BlockSpec((D, N_TARGET), lambda i: (0, 0)),
                  pl.BlockSpec((1, N_TARGET), lambda i: (0, 0))],
        out_specs=[pl.BlockSpec((BR, D), lambda i: (i, 0)),
                   pl.BlockSpec((1, D), lambda i: (0, 0)),
                   pl.BlockSpec((1, N_TARGET), lambda i: (0, 0))],
        out_shape=[jax.ShapeDtypeStruct((N_NODES, D), jnp.float32),
                   jax.ShapeDtypeStruct((1, D), jnp.float32),
                   jax.ShapeDtypeStruct((1, N_TARGET), jnp.float32)],
    )(p0, p1, cnt, b, wf, bf)


def kernel(x, H, W1, b1, W2, b2, Wf, bf):
    node_idx = H[0]
    hyedge_idx = H[1]
    zeros_d = jnp.zeros((ROWS_PER_TILE, D), jnp.float32)
    ones_c = jnp.ones((CH, CW), jnp.float32)

    counts = _sc_counts(H.reshape(2 * E), ones_c, zeros_d)
    dv = counts[:N_NODES]
    de = counts[NPAD:NPAD + N_NODES]

    y1 = _tc_matmul(x, W1)
    ep = _sc_seg_sum(y1, node_idx, hyedge_idx, zeros_d)
    e1 = _tc_combine(ep[:N_NODES], ep[NPAD:NPAD + N_NODES], de)
    np_ = _sc_seg_sum(e1, hyedge_idx, node_idx, zeros_d)
    y2 = _tc_combine_matmul(np_[:N_NODES], np_[NPAD:NPAD + N_NODES], dv,
                            b1.reshape(1, D), W2)
    ep2 = _sc_seg_sum(y2, node_idx, hyedge_idx, zeros_d)
    e2 = _tc_combine(ep2[:N_NODES], ep2[NPAD:NPAD + N_NODES], de)
    np2 = _sc_seg_sum(e2, hyedge_idx, node_idx, zeros_d)
    feats, fp, out = _tc_final(np2[:N_NODES], np2[NPAD:NPAD + N_NODES], dv,
                               b2.reshape(1, D), Wf, bf.reshape(1, N_TARGET))
    return (out.reshape(N_TARGET), feats, fp)
